# baseline (speedup 1.0000x reference)
import functools

import jax
import jax.numpy as jnp
import numpy as np
from jax import lax
from jax.experimental import pallas as pl
from jax.experimental.pallas import tpu as pltpu
from jax.experimental.pallas import tpu_sc as plsc

_N = 50000
_D = 256
_MAXP = 100000
_NW = 32
_PAIRS_PER_W = _MAXP // _NW
_CSLOT = 128
_CPAIR = _CSLOT // 2
_CHUNKS = 50
_ROWBLK = 1000


def _draws():
    key = jax.random.key(42)
    kc, k0, k1 = jax.random.split(key, 3)
    g = jax.random.gumbel(kc, (_MAXP, 2), jnp.float32)
    u0 = jax.random.uniform(k0, (2 * _MAXP,), jnp.float32)
    u1 = jax.random.uniform(k1, (2 * _MAXP,), jnp.float32)
    return g, u0, u1


_CROWS = 392
_GROWS = 782


def _aux_body(margin_ref, np_ref, emb_ref, e_ref, g0_ref, g1_ref, ep_ref,
              aux_ref, cs_ref, pi_ref, pf_ref):
    i = pl.program_id(0)
    x = emb_ref[...]
    ss = jnp.sum(x * x, axis=1, keepdims=True)
    norm = jnp.sqrt(ss)
    scale = 1.0 / jnp.maximum(norm, 1e-12)
    mg = jnp.full_like(scale, margin_ref[0, 0])
    z = jnp.zeros((x.shape[0], 124), jnp.float32)
    aux_ref[...] = jnp.concatenate([e_ref[...], scale, mg, z], axis=1)

    @pl.when(i == _N // _ROWBLK - 1)
    def _():
        g0 = g0_ref[...]
        g1 = g1_ref[...]
        ep = ep_ref[...]
        rfl = jax.lax.broadcasted_iota(jnp.int32, (_CROWS, 128), 0)
        cfl = jax.lax.broadcasted_iota(jnp.int32, (_CROWS, 128), 1)
        validm = (rfl * 128 + cfl) < _N
        w = jnp.where(validm[None], ep + 1e-6, 0.0)
        s = jnp.sum(w, axis=(1, 2), keepdims=True)
        p = w / s
        msum = jnp.sum(ep, axis=(1, 2))
        l0 = jnp.log(msum[0] / _N)
        l1 = jnp.log(msum[1] / _N)
        n0 = jnp.sum(((g0 + l0) >= (g1 + l1)).astype(jnp.int32))
        n1 = np_ref[0, 0] - n0
        p2 = p.reshape(2 * _CROWS, 128)
        ur = jax.lax.broadcasted_iota(jnp.int32, (128, 128), 0)
        uc = jax.lax.broadcasted_iota(jnp.int32, (128, 128), 1)
        U = (ur <= uc).astype(jnp.float32)
        y2 = jax.lax.dot_general(p2, U, (((1,), (0,)), ((), ())),
                                 precision=jax.lax.Precision.HIGHEST,
                                 preferred_element_type=jnp.float32)
        rs = y2[:, 127:128]
        lr = jax.lax.broadcasted_iota(jnp.int32, (2 * _CROWS, 2 * _CROWS), 0)
        lc = jax.lax.broadcasted_iota(jnp.int32, (2 * _CROWS, 2 * _CROWS), 1)
        L = ((lc < lr) & ((lc // _CROWS) == (lr // _CROWS))).astype(
            jnp.float32)
        offs = jax.lax.dot_general(L, rs, (((1,), (0,)), ((), ())),
                                   precision=jax.lax.Precision.HIGHEST,
                                   preferred_element_type=jnp.float32)
        cs2 = y2 + offs
        cs_ref[...] = cs2.reshape(2, _CROWS, 128)
        t0 = cs2[_CROWS - 1, 127]
        t1 = cs2[2 * _CROWS - 1, 127]
        pi_ref[...] = jnp.concatenate(
            [jnp.reshape(2 * n0, (1, 1)), jnp.reshape(2 * (n0 + n1), (1, 1)),
             jnp.zeros((1, 14), jnp.int32)], axis=1)
        pf_ref[...] = jnp.concatenate(
            [jnp.reshape(t0, (1, 1)), jnp.reshape(t1, (1, 1)),
             jnp.zeros((1, 14), jnp.float32)], axis=1)


def _build_aux(embeddings, E, margin, num_pairs, g0p, g1p, ep):
    grid = (_N // _ROWBLK,)
    npa = jnp.asarray(num_pairs, jnp.int32).reshape(1, 1)
    return pl.pallas_call(
        _aux_body,
        grid=grid,
        in_specs=[
            pl.BlockSpec((1, 1), lambda i: (0, 0)),
            pl.BlockSpec((1, 1), lambda i: (0, 0)),
            pl.BlockSpec((_ROWBLK, _D), lambda i: (i, 0)),
            pl.BlockSpec((_ROWBLK, 2), lambda i: (i, 0)),
            pl.BlockSpec((_GROWS, 128), lambda i: (0, 0)),
            pl.BlockSpec((_GROWS, 128), lambda i: (0, 0)),
            pl.BlockSpec((2, _CROWS, 128), lambda i: (0, 0, 0)),
        ],
        out_specs=[
            pl.BlockSpec((_ROWBLK, 128), lambda i: (i, 0)),
            pl.BlockSpec((2, _CROWS, 128), lambda i: (0, 0, 0)),
            pl.BlockSpec((1, 16), lambda i: (0, 0)),
            pl.BlockSpec((1, 16), lambda i: (0, 0)),
        ],
        out_shape=[
            jax.ShapeDtypeStruct((_N, 128), jnp.float32),
            jax.ShapeDtypeStruct((2, _CROWS, 128), jnp.float32),
            jax.ShapeDtypeStruct((1, 16), jnp.int32),
            jax.ShapeDtypeStruct((1, 16), jnp.float32),
        ],
    )(margin.reshape(1, 1), npa, embeddings, E, g0p, g1p, ep)


_UWIN = 6408
_UMAXST = 2 * _MAXP - _UWIN
_GROUPS = _CHUNKS * _CSLOT // 16


def _search_body(cs_hbm, u0_hbm, u1_hbm, pi_hbm, pf_hbm, idx_out_hbm,
                 cs_v, u0_v, u1_v, idx_v, pi_v, pf_v):
    wid = lax.axis_index("s") * 2 + lax.axis_index("c")
    base = wid * (2 * _PAIRS_PER_W)
    pltpu.sync_copy(cs_hbm, cs_v)
    pltpu.sync_copy(pi_hbm, pi_v)
    pltpu.sync_copy(pf_hbm, pf_v)
    pi = pi_v[...]
    pf = pf_v[...]
    n0_2 = pi[0]
    nvalid2 = pi[1]
    total0 = pf[0]
    total1 = pf[1]

    start0 = jnp.minimum((base // 8) * 8, _UMAXST)
    start1 = (jnp.clip(base - n0_2, 0, _UMAXST) // 8) * 8
    start0 = pl.multiple_of(start0, 8)
    start1 = pl.multiple_of(start1, 8)
    pltpu.sync_copy(u0_hbm.at[pl.ds(start0, _UWIN)], u0_v.at[0])
    pltpu.sync_copy(u1_hbm.at[pl.ds(start1, _UWIN)], u1_v.at[0])

    lane = lax.iota(jnp.int32, 16)

    def query(g):
        t = base + g * 16 + lane
        is_c0 = t < n0_2
        i0 = jnp.clip(t - start0, 0, _UWIN - 1)
        i1 = jnp.clip(jnp.clip(t - n0_2, 0, 2 * _MAXP - 1) - start1,
                      0, _UWIN - 1)
        zz = jnp.zeros((16,), jnp.int32)
        uv0 = plsc.load_gather(u0_v, [zz, i0])
        uv1 = plsc.load_gather(u1_v, [zz, i1])
        u = jnp.where(is_c0, uv0, uv1)
        total = jnp.where(is_c0, total0, total1)
        r = total * (1.0 - u)
        cls = jnp.where(is_c0, 0, 1)
        return t, r, cls

    def group_body(gg, carry):
        ga = 2 * gg
        gb = 2 * gg + 1
        ta, ra, ca = query(ga)
        tb, rb, cb = query(gb)

        lo_a = jnp.zeros((16,), jnp.int32)
        hi_a = jnp.full((16,), _N, jnp.int32)
        lo_b = jnp.zeros((16,), jnp.int32)
        hi_b = jnp.full((16,), _N, jnp.int32)

        def step(_, st):
            lo_a, hi_a, lo_b, hi_b = st
            mid_a = (lo_a + hi_a) >> 1
            mid_b = (lo_b + hi_b) >> 1
            va = plsc.load_gather(cs_v, [ca, mid_a >> 7, mid_a & 127])
            vb = plsc.load_gather(cs_v, [cb, mid_b >> 7, mid_b & 127])
            ba = va < ra
            bb = vb < rb
            return (jnp.where(ba, mid_a + 1, lo_a),
                    jnp.where(ba, hi_a, mid_a),
                    jnp.where(bb, mid_b + 1, lo_b),
                    jnp.where(bb, hi_b, mid_b))

        lo_a, hi_a, lo_b, hi_b = lax.fori_loop(
            0, 16, step, (lo_a, hi_a, lo_b, hi_b))
        idx_a = jnp.where(ta < nvalid2, jnp.clip(lo_a, 0, _N - 1), 0)
        idx_b = jnp.where(tb < nvalid2, jnp.clip(lo_b, 0, _N - 1), 0)
        idx_v[pl.ds(ga * 16, 16)] = idx_a
        idx_v[pl.ds(gb * 16, 16)] = idx_b
        return carry

    lax.fori_loop(0, _GROUPS // 2, group_body, 0)
    pltpu.sync_copy(idx_v, idx_out_hbm.at[wid])


def _search_indices(cs01, u0, u1, pi, pf):
    mesh = plsc.VectorSubcoreMesh(core_axis_name="c", subcore_axis_name="s")
    fn = pl.kernel(
        _search_body,
        out_type=jax.ShapeDtypeStruct((_NW, _CHUNKS * _CSLOT), jnp.int32),
        mesh=mesh,
        scratch_types=[
            pltpu.VMEM((2, _CROWS, 128), jnp.float32),
            pltpu.VMEM((1, _UWIN), jnp.float32),
            pltpu.VMEM((1, _UWIN), jnp.float32),
            pltpu.VMEM((_CHUNKS * _CSLOT,), jnp.int32),
            pltpu.VMEM((16,), jnp.int32),
            pltpu.VMEM((16,), jnp.float32),
        ],
        compiler_params=pltpu.CompilerParams(needs_layout_passes=False),
    )
    return fn(cs01, u0, u1, pi, pf)


def _loss_body(emb_hbm, aux_hbm, idx_hbm, out_hbm, idx_v, rows0, auxr0,
               rows1, auxr1, out_v, sem0e, sem0a, sem1e, sem1a):
    wid = lax.axis_index("s") * 2 + lax.axis_index("c")
    pltpu.sync_copy(idx_hbm.at[wid], idx_v)
    lane = lax.iota(jnp.int32, 16)

    def issue(c, rows_v, auxr_v, seme, sema):
        pltpu.async_copy(emb_hbm.at[idx_v.at[c]], rows_v, seme)
        pltpu.async_copy(aux_hbm.at[idx_v.at[c]], auxr_v, sema)

    def drain(rows_v, auxr_v, seme, sema):
        pltpu.make_async_copy(emb_hbm.at[idx_v.at[0]], rows_v, seme).wait()
        pltpu.make_async_copy(aux_hbm.at[idx_v.at[0]], auxr_v, sema).wait()

    def compute(c, rows_v, auxr_v, total):
        def one_pair(p):
            i = 2 * p
            j = 2 * p + 1
            acc = rows_v[i, pl.ds(0, 16)] * rows_v[j, pl.ds(0, 16)]
            for k in range(1, _D // 16):
                acc = acc + (rows_v[i, pl.ds(16 * k, 16)] *
                             rows_v[j, pl.ds(16 * k, 16)])
            dot = acc[0]
            for k in range(1, 16):
                dot = dot + acc[k]
            ap = auxr_v[i] * auxr_v[j]
            cos = dot * ap[2]
            post = ap[0] + ap[1]
            mg = auxr_v[i][3]
            dpos = cos - 1.0
            dneg = jnp.maximum(cos - mg, 0.0)
            pls = dpos * dpos * post + dneg * dneg * (1.0 - post)
            valid = (c * _CPAIR + p) < _PAIRS_PER_W
            return jnp.where(valid, pls, jnp.float32(0.0))

        def pair_body(pp, t):
            return t + one_pair(2 * pp) + one_pair(2 * pp + 1)

        return lax.fori_loop(0, _CPAIR // 2, pair_body, total)

    issue(0, rows0, auxr0, sem0e, sem0a)
    issue(1, rows1, auxr1, sem1e, sem1a)

    def dstep(cc, total):
        c0 = 2 * cc
        drain(rows0, auxr0, sem0e, sem0a)
        total = compute(c0, rows0, auxr0, total)

        @pl.when(cc < _CHUNKS // 2 - 1)
        def _():
            issue(c0 + 2, rows0, auxr0, sem0e, sem0a)

        drain(rows1, auxr1, sem1e, sem1a)
        total = compute(c0 + 1, rows1, auxr1, total)

        @pl.when(cc < _CHUNKS // 2 - 1)
        def _():
            issue(c0 + 3, rows1, auxr1, sem1e, sem1a)

        return total

    total = lax.fori_loop(0, _CHUNKS // 2, dstep, jnp.float32(0.0))
    out_v[...] = jnp.where(lane == 0, total, jnp.float32(0.0))
    pltpu.sync_copy(out_v, out_hbm.at[wid])


def _pair_loss_sums(embeddings, aux, idx3):
    mesh = plsc.VectorSubcoreMesh(core_axis_name="c", subcore_axis_name="s")
    fn = pl.kernel(
        _loss_body,
        out_type=jax.ShapeDtypeStruct((_NW, 16), jnp.float32),
        mesh=mesh,
        scratch_types=[
            pltpu.VMEM((_CHUNKS, _CSLOT), jnp.int32),
            pltpu.VMEM((_CSLOT, _D), jnp.float32),
            pltpu.VMEM((_CSLOT, 128), jnp.float32),
            pltpu.VMEM((_CSLOT, _D), jnp.float32),
            pltpu.VMEM((_CSLOT, 128), jnp.float32),
            pltpu.VMEM((16,), jnp.float32),
            pltpu.SemaphoreType.DMA,
            pltpu.SemaphoreType.DMA,
            pltpu.SemaphoreType.DMA,
            pltpu.SemaphoreType.DMA,
        ],
    )
    return fn(embeddings, aux, idx3)


def kernel(embeddings, E, raw_margin, num_pairs):
    g, u0, u1 = _draws()
    margin = jax.nn.softplus(raw_margin)
    ep = jnp.pad(E, ((0, _CROWS * 128 - _N), (0, 0))).T.reshape(
        2, _CROWS, 128)
    gpad = _GROWS * 128 - _MAXP
    g0p = jnp.pad(g[:, 0], (0, gpad), constant_values=-1e30).reshape(
        _GROWS, 128)
    g1p = jnp.pad(g[:, 1], (0, gpad)).reshape(_GROWS, 128)
    aux, cs3, pi, pf = _build_aux(embeddings, E, margin, num_pairs,
                                  g0p, g1p, ep)

    idx2 = _search_indices(cs3, u0, u1, pi.reshape(16), pf.reshape(16))
    idx3 = idx2.reshape(_NW, _CHUNKS, _CSLOT)

    sums = _pair_loss_sums(embeddings, aux, idx3)
    return jnp.sum(sums) / _MAXP


# speedup vs baseline: 1.1632x; 1.1632x over previous
import functools

import jax
import jax.numpy as jnp
import numpy as np
from jax import lax
from jax.experimental import pallas as pl
from jax.experimental.pallas import tpu as pltpu
from jax.experimental.pallas import tpu_sc as plsc

_N = 50000
_D = 256
_MAXP = 100000
_NW = 32
_PAIRS_PER_W = _MAXP // _NW
_CSLOT = 128
_CPAIR = _CSLOT // 2
_CHUNKS = 50
_ROWBLK = 1000


def _draws():
    key = jax.random.key(42)
    kc, k0, k1 = jax.random.split(key, 3)
    g = jax.random.gumbel(kc, (_MAXP, 2), jnp.float32)
    u0 = jax.random.uniform(k0, (2 * _MAXP,), jnp.float32)
    u1 = jax.random.uniform(k1, (2 * _MAXP,), jnp.float32)
    return g, u0, u1


_CROWS = 392
_GROWS = 782


def _aux_body(margin_ref, emb_ref, e_ref, aux_ref):
    x = emb_ref[...]
    ss = jnp.sum(x * x, axis=1, keepdims=True)
    norm = jnp.sqrt(ss)
    scale = 1.0 / jnp.maximum(norm, 1e-12)
    mg = jnp.full_like(scale, margin_ref[0, 0])
    z = jnp.zeros((x.shape[0], 124), jnp.float32)
    aux_ref[...] = jnp.concatenate([e_ref[...], scale, mg, z], axis=1)


def _cdf_body(np_ref, g0_ref, g1_ref, ep_ref, cs_ref, pi_ref, pf_ref):
    if True:
        g0 = g0_ref[...]
        g1 = g1_ref[...]
        ep = ep_ref[...]
        rfl = jax.lax.broadcasted_iota(jnp.int32, (_CROWS, 128), 0)
        cfl = jax.lax.broadcasted_iota(jnp.int32, (_CROWS, 128), 1)
        validm = (rfl * 128 + cfl) < _N
        w = jnp.where(validm[None], ep + 1e-6, 0.0)
        s = jnp.sum(w, axis=(1, 2), keepdims=True)
        p = w / s
        msum = jnp.sum(ep, axis=(1, 2))
        l0 = jnp.log(msum[0] / _N)
        l1 = jnp.log(msum[1] / _N)
        n0 = jnp.sum(((g0 + l0) >= (g1 + l1)).astype(jnp.int32))
        n1 = np_ref[0, 0] - n0
        p2 = p.reshape(2 * _CROWS, 128)
        ur = jax.lax.broadcasted_iota(jnp.int32, (128, 128), 0)
        uc = jax.lax.broadcasted_iota(jnp.int32, (128, 128), 1)
        U = (ur <= uc).astype(jnp.float32)
        y2 = jax.lax.dot_general(p2, U, (((1,), (0,)), ((), ())),
                                 precision=jax.lax.Precision.HIGHEST,
                                 preferred_element_type=jnp.float32)
        rs = y2[:, 127:128]
        lr = jax.lax.broadcasted_iota(jnp.int32, (2 * _CROWS, 2 * _CROWS), 0)
        lc = jax.lax.broadcasted_iota(jnp.int32, (2 * _CROWS, 2 * _CROWS), 1)
        L = ((lc < lr) & ((lc // _CROWS) == (lr // _CROWS))).astype(
            jnp.float32)
        offs = jax.lax.dot_general(L, rs, (((1,), (0,)), ((), ())),
                                   precision=jax.lax.Precision.HIGHEST,
                                   preferred_element_type=jnp.float32)
        cs2 = y2 + offs
        cs_ref[...] = cs2.reshape(2, _CROWS, 128)
        t0 = cs2[_CROWS - 1, 127]
        t1 = cs2[2 * _CROWS - 1, 127]
        pi_ref[...] = jnp.concatenate(
            [jnp.reshape(2 * n0, (1, 1)), jnp.reshape(2 * (n0 + n1), (1, 1)),
             jnp.zeros((1, 14), jnp.int32)], axis=1)
        pf_ref[...] = jnp.concatenate(
            [jnp.reshape(t0, (1, 1)), jnp.reshape(t1, (1, 1)),
             jnp.zeros((1, 14), jnp.float32)], axis=1)


def _build_aux(embeddings, E, margin):
    grid = (_N // _ROWBLK,)
    return pl.pallas_call(
        _aux_body,
        grid=grid,
        in_specs=[
            pl.BlockSpec((1, 1), lambda i: (0, 0)),
            pl.BlockSpec((_ROWBLK, _D), lambda i: (i, 0)),
            pl.BlockSpec((_ROWBLK, 2), lambda i: (i, 0)),
        ],
        out_specs=pl.BlockSpec((_ROWBLK, 128), lambda i: (i, 0)),
        out_shape=jax.ShapeDtypeStruct((_N, 128), jnp.float32),
    )(margin.reshape(1, 1), embeddings, E)


def _build_cdf(num_pairs, g0p, g1p, ep):
    npa = jnp.asarray(num_pairs, jnp.int32).reshape(1, 1)
    return pl.pallas_call(
        _cdf_body,
        in_specs=[
            pl.BlockSpec((1, 1), lambda: (0, 0)),
            pl.BlockSpec((_GROWS, 128), lambda: (0, 0)),
            pl.BlockSpec((_GROWS, 128), lambda: (0, 0)),
            pl.BlockSpec((2, _CROWS, 128), lambda: (0, 0, 0)),
        ],
        out_specs=[
            pl.BlockSpec((2, _CROWS, 128), lambda: (0, 0, 0)),
            pl.BlockSpec((1, 16), lambda: (0, 0)),
            pl.BlockSpec((1, 16), lambda: (0, 0)),
        ],
        out_shape=[
            jax.ShapeDtypeStruct((2, _CROWS, 128), jnp.float32),
            jax.ShapeDtypeStruct((1, 16), jnp.int32),
            jax.ShapeDtypeStruct((1, 16), jnp.float32),
        ],
    )(npa, g0p, g1p, ep)


_UWIN = 6408
_UMAXST = 2 * _MAXP - _UWIN
_GROUPS = _CHUNKS * _CSLOT // 16


def _search_body(cs_hbm, u0_hbm, u1_hbm, pi_hbm, pf_hbm, idx_out_hbm,
                 cs_v, u0_v, u1_v, idx_v, pi_v, pf_v):
    wid = lax.axis_index("s") * 2 + lax.axis_index("c")
    base = wid * (2 * _PAIRS_PER_W)
    pltpu.sync_copy(cs_hbm, cs_v)
    pltpu.sync_copy(pi_hbm, pi_v)
    pltpu.sync_copy(pf_hbm, pf_v)
    pi = pi_v[...]
    pf = pf_v[...]
    n0_2 = pi[0]
    nvalid2 = pi[1]
    total0 = pf[0]
    total1 = pf[1]

    start0 = jnp.minimum((base // 8) * 8, _UMAXST)
    start1 = (jnp.clip(base - n0_2, 0, _UMAXST) // 8) * 8
    start0 = pl.multiple_of(start0, 8)
    start1 = pl.multiple_of(start1, 8)
    pltpu.sync_copy(u0_hbm.at[pl.ds(start0, _UWIN)], u0_v.at[0])
    pltpu.sync_copy(u1_hbm.at[pl.ds(start1, _UWIN)], u1_v.at[0])

    lane = lax.iota(jnp.int32, 16)

    def query(g):
        t = base + g * 16 + lane
        is_c0 = t < n0_2
        i0 = jnp.clip(t - start0, 0, _UWIN - 1)
        i1 = jnp.clip(jnp.clip(t - n0_2, 0, 2 * _MAXP - 1) - start1,
                      0, _UWIN - 1)
        zz = jnp.zeros((16,), jnp.int32)
        uv0 = plsc.load_gather(u0_v, [zz, i0])
        uv1 = plsc.load_gather(u1_v, [zz, i1])
        u = jnp.where(is_c0, uv0, uv1)
        total = jnp.where(is_c0, total0, total1)
        r = total * (1.0 - u)
        cls = jnp.where(is_c0, 0, 1)
        return t, r, cls

    def group_body(gg, carry):
        ga = 2 * gg
        gb = 2 * gg + 1
        ta, ra, ca = query(ga)
        tb, rb, cb = query(gb)

        lo_a = jnp.zeros((16,), jnp.int32)
        hi_a = jnp.full((16,), _N, jnp.int32)
        lo_b = jnp.zeros((16,), jnp.int32)
        hi_b = jnp.full((16,), _N, jnp.int32)

        def step(_, st):
            lo_a, hi_a, lo_b, hi_b = st
            mid_a = (lo_a + hi_a) >> 1
            mid_b = (lo_b + hi_b) >> 1
            va = plsc.load_gather(cs_v, [ca, mid_a >> 7, mid_a & 127])
            vb = plsc.load_gather(cs_v, [cb, mid_b >> 7, mid_b & 127])
            ba = va < ra
            bb = vb < rb
            return (jnp.where(ba, mid_a + 1, lo_a),
                    jnp.where(ba, hi_a, mid_a),
                    jnp.where(bb, mid_b + 1, lo_b),
                    jnp.where(bb, hi_b, mid_b))

        lo_a, hi_a, lo_b, hi_b = lax.fori_loop(
            0, 16, step, (lo_a, hi_a, lo_b, hi_b))
        idx_a = jnp.where(ta < nvalid2, jnp.clip(lo_a, 0, _N - 1), 0)
        idx_b = jnp.where(tb < nvalid2, jnp.clip(lo_b, 0, _N - 1), 0)
        idx_v[pl.ds(ga * 16, 16)] = idx_a
        idx_v[pl.ds(gb * 16, 16)] = idx_b
        return carry

    lax.fori_loop(0, _GROUPS // 2, group_body, 0)
    pltpu.sync_copy(idx_v, idx_out_hbm.at[wid])


def _search_indices(cs01, u0, u1, pi, pf):
    mesh = plsc.VectorSubcoreMesh(core_axis_name="c", subcore_axis_name="s")
    fn = pl.kernel(
        _search_body,
        out_type=jax.ShapeDtypeStruct((_NW, _CHUNKS * _CSLOT), jnp.int32),
        mesh=mesh,
        scratch_types=[
            pltpu.VMEM((2, _CROWS, 128), jnp.float32),
            pltpu.VMEM((1, _UWIN), jnp.float32),
            pltpu.VMEM((1, _UWIN), jnp.float32),
            pltpu.VMEM((_CHUNKS * _CSLOT,), jnp.int32),
            pltpu.VMEM((16,), jnp.int32),
            pltpu.VMEM((16,), jnp.float32),
        ],
        compiler_params=pltpu.CompilerParams(needs_layout_passes=False),
    )
    return fn(cs01, u0, u1, pi, pf)


def _loss_body(emb_hbm, aux_hbm, idx_hbm, out_hbm, idx_v, rows0, auxr0,
               rows1, auxr1, out_v, sem0e, sem0a, sem1e, sem1a):
    wid = lax.axis_index("s") * 2 + lax.axis_index("c")
    pltpu.sync_copy(idx_hbm.at[wid], idx_v)
    lane = lax.iota(jnp.int32, 16)

    def issue(c, rows_v, auxr_v, seme, sema):
        pltpu.async_copy(emb_hbm.at[idx_v.at[c]], rows_v, seme)
        pltpu.async_copy(aux_hbm.at[idx_v.at[c]], auxr_v, sema)

    def drain(rows_v, auxr_v, seme, sema):
        pltpu.make_async_copy(emb_hbm.at[idx_v.at[0]], rows_v, seme).wait()
        pltpu.make_async_copy(aux_hbm.at[idx_v.at[0]], auxr_v, sema).wait()

    def compute(c, rows_v, auxr_v, total):
        def one_pair(p):
            i = 2 * p
            j = 2 * p + 1
            acc = rows_v[i, pl.ds(0, 16)] * rows_v[j, pl.ds(0, 16)]
            for k in range(1, _D // 16):
                acc = acc + (rows_v[i, pl.ds(16 * k, 16)] *
                             rows_v[j, pl.ds(16 * k, 16)])
            dot = acc[0]
            for k in range(1, 16):
                dot = dot + acc[k]
            ap = auxr_v[i] * auxr_v[j]
            cos = dot * ap[2]
            post = ap[0] + ap[1]
            mg = auxr_v[i][3]
            dpos = cos - 1.0
            dneg = jnp.maximum(cos - mg, 0.0)
            pls = dpos * dpos * post + dneg * dneg * (1.0 - post)
            valid = (c * _CPAIR + p) < _PAIRS_PER_W
            return jnp.where(valid, pls, jnp.float32(0.0))

        def pair_body(pp, t):
            return t + one_pair(2 * pp) + one_pair(2 * pp + 1)

        return lax.fori_loop(0, _CPAIR // 2, pair_body, total)

    issue(0, rows0, auxr0, sem0e, sem0a)
    issue(1, rows1, auxr1, sem1e, sem1a)

    def dstep(cc, total):
        c0 = 2 * cc
        drain(rows0, auxr0, sem0e, sem0a)
        total = compute(c0, rows0, auxr0, total)

        @pl.when(cc < _CHUNKS // 2 - 1)
        def _():
            issue(c0 + 2, rows0, auxr0, sem0e, sem0a)

        drain(rows1, auxr1, sem1e, sem1a)
        total = compute(c0 + 1, rows1, auxr1, total)

        @pl.when(cc < _CHUNKS // 2 - 1)
        def _():
            issue(c0 + 3, rows1, auxr1, sem1e, sem1a)

        return total

    total = lax.fori_loop(0, _CHUNKS // 2, dstep, jnp.float32(0.0))
    out_v[...] = jnp.where(lane == 0, total, jnp.float32(0.0))
    pltpu.sync_copy(out_v, out_hbm.at[wid])


def _pair_loss_sums(embeddings, aux, idx3):
    mesh = plsc.VectorSubcoreMesh(core_axis_name="c", subcore_axis_name="s")
    fn = pl.kernel(
        _loss_body,
        out_type=jax.ShapeDtypeStruct((_NW, 16), jnp.float32),
        mesh=mesh,
        scratch_types=[
            pltpu.VMEM((_CHUNKS, _CSLOT), jnp.int32),
            pltpu.VMEM((_CSLOT, _D), jnp.float32),
            pltpu.VMEM((_CSLOT, 128), jnp.float32),
            pltpu.VMEM((_CSLOT, _D), jnp.float32),
            pltpu.VMEM((_CSLOT, 128), jnp.float32),
            pltpu.VMEM((16,), jnp.float32),
            pltpu.SemaphoreType.DMA,
            pltpu.SemaphoreType.DMA,
            pltpu.SemaphoreType.DMA,
            pltpu.SemaphoreType.DMA,
        ],
    )
    return fn(embeddings, aux, idx3)


def kernel(embeddings, E, raw_margin, num_pairs):
    g, u0, u1 = _draws()
    margin = jax.nn.softplus(raw_margin)
    ep = jnp.pad(E, ((0, _CROWS * 128 - _N), (0, 0))).T.reshape(
        2, _CROWS, 128)
    gpad = _GROWS * 128 - _MAXP
    g0p = jnp.pad(g[:, 0], (0, gpad), constant_values=-1e30).reshape(
        _GROWS, 128)
    g1p = jnp.pad(g[:, 1], (0, gpad)).reshape(_GROWS, 128)
    cs3, pi, pf = _build_cdf(num_pairs, g0p, g1p, ep)
    aux = _build_aux(embeddings, E, margin)

    idx2 = _search_indices(cs3, u0, u1, pi.reshape(16), pf.reshape(16))
    idx3 = idx2.reshape(_NW, _CHUNKS, _CSLOT)

    sums = _pair_loss_sums(embeddings, aux, idx3)
    return jnp.sum(sums) / _MAXP
